# baseline (speedup 1.0000x reference)
import functools

import jax
import jax.numpy as jnp
from jax.experimental import pallas as pl
from jax.experimental.pallas import tpu as pltpu

N = 10000
NFEAT = 256
NOUT = 256
BM = 256
BS = 2000


def _support_kernel(x_ref, w_ref, o_ref):
    s = jax.lax.dot_general(
        x_ref[...], w_ref[...],
        dimension_numbers=(((1,), (0,)), ((), ())),
        precision=jax.lax.Precision.HIGHEST,
        preferred_element_type=jnp.float32,
    )
    o_ref[...] = s.astype(jnp.bfloat16)


def _gcn_kernel(adj_ref, s_ref, b_ref, o_ref):
    a = adj_ref[...].astype(jnp.bfloat16)
    acc = jax.lax.dot_general(
        a, s_ref[...],
        dimension_numbers=(((1,), (0,)), ((), ())),
        preferred_element_type=jnp.float32,
    )
    o_ref[...] = jnp.maximum(acc + b_ref[...], 0.0)


@jax.jit
def kernel(x, adj, W, b):
    support = pl.pallas_call(
        _support_kernel,
        grid=(N // BS,),
        out_shape=jax.ShapeDtypeStruct((N, NOUT), jnp.bfloat16),
        in_specs=[
            pl.BlockSpec((BS, NFEAT), lambda i: (i, 0)),
            pl.BlockSpec((NFEAT, NOUT), lambda i: (0, 0)),
        ],
        out_specs=pl.BlockSpec((BS, NOUT), lambda i: (i, 0)),
        compiler_params=pltpu.CompilerParams(
            dimension_semantics=("arbitrary",),
        ),
    )(x, W)

    b2 = b.reshape(1, NOUT)
    num_m = pl.cdiv(N, BM)
    out = pl.pallas_call(
        _gcn_kernel,
        grid=(num_m,),
        out_shape=jax.ShapeDtypeStruct((N, NOUT), jnp.float32),
        in_specs=[
            pl.BlockSpec((BM, N), lambda i: (i, 0)),
            pl.BlockSpec((N, NOUT), lambda i: (0, 0)),
            pl.BlockSpec((1, NOUT), lambda i: (0, 0)),
        ],
        out_specs=pl.BlockSpec((BM, NOUT), lambda i: (i, 0)),
        compiler_params=pltpu.CompilerParams(
            dimension_semantics=("arbitrary",),
            vmem_limit_bytes=56 * 1024 * 1024,
        ),
    )(adj, support, b2)
    return out


# speedup vs baseline: 1.0308x; 1.0308x over previous
import jax
import jax.numpy as jnp
from jax.experimental import pallas as pl
from jax.experimental.pallas import tpu as pltpu

N = 10000
NFEAT = 256
NOUT = 256
BM = 256
BS = 2000


def _gcn_kernel(adj_ref, x_ref, w_ref, b_ref, o_ref, s_ref):
    @pl.when(pl.program_id(0) == 0)
    def _():
        for c in range(N // BS):
            s = jax.lax.dot_general(
                x_ref[pl.ds(c * BS, BS), :], w_ref[...],
                dimension_numbers=(((1,), (0,)), ((), ())),
                precision=jax.lax.Precision.HIGHEST,
                preferred_element_type=jnp.float32,
            )
            s_ref[pl.ds(c * BS, BS), :] = s.astype(jnp.bfloat16)

    a = adj_ref[...].astype(jnp.bfloat16)
    acc = jax.lax.dot_general(
        a, s_ref[...],
        dimension_numbers=(((1,), (0,)), ((), ())),
        preferred_element_type=jnp.float32,
    )
    o_ref[...] = jnp.maximum(acc + b_ref[...], 0.0)


@jax.jit
def kernel(x, adj, W, b):
    b2 = b.reshape(1, NOUT)
    num_m = pl.cdiv(N, BM)
    return pl.pallas_call(
        _gcn_kernel,
        grid=(num_m,),
        out_shape=jax.ShapeDtypeStruct((N, NOUT), jnp.float32),
        in_specs=[
            pl.BlockSpec((BM, N), lambda i: (i, 0)),
            pl.BlockSpec((N, NFEAT), lambda i: (0, 0)),
            pl.BlockSpec((NFEAT, NOUT), lambda i: (0, 0)),
            pl.BlockSpec((1, NOUT), lambda i: (0, 0)),
        ],
        out_specs=pl.BlockSpec((BM, NOUT), lambda i: (i, 0)),
        scratch_shapes=[pltpu.VMEM((N, NOUT), jnp.bfloat16)],
        compiler_params=pltpu.CompilerParams(
            dimension_semantics=("arbitrary",),
            vmem_limit_bytes=56 * 1024 * 1024,
        ),
    )(adj, x, W, b2)


# speedup vs baseline: 1.0849x; 1.0525x over previous
import jax
import jax.numpy as jnp
from jax.experimental import pallas as pl
from jax.experimental.pallas import tpu as pltpu

N = 10000
NFEAT = 256
NOUT = 256
BM = 256
BS = 2000


def _gcn_kernel(adj_ref, x_ref, w_ref, b_ref, o_ref, s_ref):
    @pl.when(pl.program_id(0) == 0)
    def _():
        for c in range(N // BS):
            s = jax.lax.dot_general(
                x_ref[pl.ds(c * BS, BS), :], w_ref[...],
                dimension_numbers=(((1,), (0,)), ((), ())),
                precision=jax.lax.Precision.DEFAULT,
                preferred_element_type=jnp.float32,
            )
            s_ref[pl.ds(c * BS, BS), :] = s.astype(jnp.bfloat16)

    a = adj_ref[...].astype(jnp.bfloat16)
    acc = jax.lax.dot_general(
        a, s_ref[...],
        dimension_numbers=(((1,), (0,)), ((), ())),
        preferred_element_type=jnp.float32,
    )
    o_ref[...] = jnp.maximum(acc + b_ref[...], 0.0)


@jax.jit
def kernel(x, adj, W, b):
    b2 = b.reshape(1, NOUT)
    num_m = pl.cdiv(N, BM)
    return pl.pallas_call(
        _gcn_kernel,
        grid=(num_m,),
        out_shape=jax.ShapeDtypeStruct((N, NOUT), jnp.float32),
        in_specs=[
            pl.BlockSpec((BM, N), lambda i: (i, 0)),
            pl.BlockSpec((N, NFEAT), lambda i: (0, 0)),
            pl.BlockSpec((NFEAT, NOUT), lambda i: (0, 0)),
            pl.BlockSpec((1, NOUT), lambda i: (0, 0)),
        ],
        out_specs=pl.BlockSpec((BM, NOUT), lambda i: (i, 0)),
        scratch_shapes=[pltpu.VMEM((N, NOUT), jnp.bfloat16)],
        compiler_params=pltpu.CompilerParams(
            dimension_semantics=("arbitrary",),
            vmem_limit_bytes=56 * 1024 * 1024,
        ),
    )(adj, x, W, b2)
